# Initial kernel scaffold; baseline (speedup 1.0000x reference)
import jax
import jax.numpy as jnp
from jax.experimental import pallas as pl


def kernel(input_ids, token_type_ids, word_embeddings, token_type_embeddings, ln_gamma, ln_beta):
    raise NotImplementedError("write your pallas kernel here")


# speedup vs baseline: 1.6392x; 1.6392x over previous
import functools

import jax
import jax.numpy as jnp
from jax import lax
from jax.experimental import pallas as pl
from jax.experimental.pallas import tpu as pltpu
from jax.experimental.pallas import tpu_sc as plsc

D = 64
L = 16
NK = D // L
NC, NS = 2, 16
NW = NC * NS
EPS = 1e-12

_MAGIC = 0x5F3759DF


def _rsqrt16(v):
    i = plsc.bitcast(v, jnp.int32)
    y = plsc.bitcast(_MAGIC - (i >> 1), jnp.float32)
    half = v * 0.5
    y = y * (1.5 - half * y * y)
    y = y * (1.5 - half * y * y)
    return y


def _lanesum(v):
    return jnp.broadcast_to(jnp.sum(v), (L,))


def _sc_body(nch, c, ids, tti, table, tt, gamma, beta, out,
             idx_v, tti_v, din, dout, ttv, gb, sem_g, sem_o):
    wid = lax.axis_index("s") * NC + lax.axis_index("c")
    rpw = nch * c
    base = wid * rpw

    pltpu.sync_copy(ids.at[wid], idx_v)
    pltpu.sync_copy(tti.at[wid], tti_v)

    def issue_gathers(g, b):
        for i in range(c // L):
            ivec = idx_v[g, pl.ds(i * L, L)]
            for j in range(L):
                r = i * L + j
                pltpu.make_async_copy(
                    table.at[pl.ds(ivec[j], 1)],
                    din.at[b].at[pl.ds(r, 1)],
                    sem_g.at[b],
                ).start()

    def drain_gathers(b):
        pltpu.make_async_copy(
            table.at[pl.ds(0, c)], din.at[b], sem_g.at[b]
        ).wait()

    issue_gathers(0, 0)
    issue_gathers(1, 1)

    pltpu.sync_copy(gamma, gb.at[0])
    pltpu.sync_copy(beta, gb.at[1])
    pltpu.sync_copy(tt, ttv)
    gvec = [gb[0, pl.ds(k * L, L)] for k in range(NK)]
    bvec = [gb[1, pl.ds(k * L, L)] for k in range(NK)]

    def compute_chunk(g, b):
        @plsc.parallel_loop(0, c // L, 1)
        def _rowgroup(i):
            tvec = tti_v[g, pl.ds(i * L, L)]
            for j in range(L):
                r = i * L + j
                t = tvec[j]
                u = [din[b, r, pl.ds(k * L, L)] + ttv[t, pl.ds(k * L, L)]
                     for k in range(NK)]
                s = (u[0] + u[1]) + (u[2] + u[3])
                q = [x * x for x in u]
                sq = (q[0] + q[1]) + (q[2] + q[3])
                mu = _lanesum(s) * (1.0 / D)
                msq = _lanesum(sq) * (1.0 / D)
                var = msq - mu * mu
                rinv = _rsqrt16(var + EPS)
                for k in range(NK):
                    dout[b, r, pl.ds(k * L, L)] = (u[k] - mu) * (rinv * gvec[k]) + bvec[k]

    def loop_body(g2, _):
        for b in range(2):
            g = 2 * g2 + b
            drain_gathers(b)

            @pl.when(g2 > 0)
            def _():
                pltpu.make_async_copy(
                    dout.at[b], out.at[pl.ds(base + (g - 2) * c, c)], sem_o.at[b]
                ).wait()

            compute_chunk(g, b)
            pltpu.make_async_copy(
                dout.at[b], out.at[pl.ds(base + g * c, c)], sem_o.at[b]
            ).start()

            @pl.when(g2 < (nch // 2 - 1))
            def _():
                issue_gathers(g + 2, b)

        return 0

    lax.fori_loop(0, nch // 2, loop_body, 0)

    for b in range(2):
        g = nch - 2 + b
        pltpu.make_async_copy(
            dout.at[b], out.at[pl.ds(base + g * c, c)], sem_o.at[b]
        ).wait()


def kernel(input_ids, token_type_ids, word_embeddings, token_type_embeddings,
           ln_gamma, ln_beta):
    b_, s_ = input_ids.shape
    n = b_ * s_
    rpw = n // NW
    c = 128
    nch = rpw // c

    ids = input_ids.reshape(NW, nch, c).astype(jnp.int32)
    tti = token_type_ids.reshape(NW, nch, c).astype(jnp.int32)

    body = functools.partial(_sc_body, nch, c)
    run = pl.kernel(
        body,
        out_type=jax.ShapeDtypeStruct((n, D), jnp.float32),
        mesh=plsc.VectorSubcoreMesh(core_axis_name="c", subcore_axis_name="s"),
        compiler_params=pltpu.CompilerParams(needs_layout_passes=False),
        scratch_types=[
            pltpu.VMEM((nch, c), jnp.int32),
            pltpu.VMEM((nch, c), jnp.int32),
            pltpu.VMEM((2, c, D), jnp.float32),
            pltpu.VMEM((2, c, D), jnp.float32),
            pltpu.VMEM((2, D), jnp.float32),
            pltpu.VMEM((2, D), jnp.float32),
            pltpu.SemaphoreType.DMA((2,)),
            pltpu.SemaphoreType.DMA((2,)),
        ],
    )
    out = run(ids, tti, word_embeddings, token_type_embeddings, ln_gamma, ln_beta)
    return out.reshape(b_, s_, D)
